# Initial kernel scaffold; baseline (speedup 1.0000x reference)
import jax
import jax.numpy as jnp
from jax.experimental import pallas as pl


def kernel(bag, offsets, v, u_weight, v_weight):
    raise NotImplementedError("write your pallas kernel here")


# speedup vs baseline: 16.8635x; 16.8635x over previous
import functools

import jax
import jax.numpy as jnp
from jax import lax
from jax.experimental import pallas as pl
from jax.experimental.pallas import tpu as pltpu
from jax.experimental.pallas import tpu_sc as plsc

B = 16384
D = 32
TOTAL = 131072
NSAMP = 6
NC, NS = 2, 16
NW = NC * NS
CHUNK = 128
POS_PER_W = TOTAL // NW
NCHUNK = POS_PER_W // CHUNK
VTOT = B * NSAMP
V_PER_W = VTOT // NW
NVCHUNK = V_PER_W // CHUNK
SEG_PER_TILE = B // NS


def _sc_kernel(bag2d, offsets, v2d, u_weight, v_weight):
  mesh = plsc.VectorSubcoreMesh(core_axis_name="c", subcore_axis_name="s")

  @functools.partial(
      pl.kernel,
      out_type=(
          jax.ShapeDtypeStruct((NC, B, D), jnp.float32),
          jax.ShapeDtypeStruct((NC, B), jnp.float32),
          jax.ShapeDtypeStruct((VTOT, D), jnp.float32),
      ),
      mesh=mesh,
      compiler_params=pltpu.CompilerParams(needs_layout_passes=False,
                                           use_tc_tiling_on_sc=False),
      scratch_types=(
          pltpu.VMEM((B,), jnp.int32),
          pltpu.VMEM((NCHUNK, CHUNK), jnp.int32),
          pltpu.VMEM((NCHUNK, CHUNK), jnp.int32),
          pltpu.VMEM((NVCHUNK, CHUNK), jnp.int32),
          pltpu.VMEM((CHUNK, D), jnp.float32),
          pltpu.VMEM((CHUNK, D), jnp.float32),
          pltpu.VMEM((CHUNK,), jnp.float32),
          pltpu.VMEM((256, D), jnp.float32),
          pltpu.VMEM((SEG_PER_TILE,), jnp.float32),
          pltpu.VMEM_SHARED((B, D), jnp.float32),
          pltpu.VMEM_SHARED((B,), jnp.float32),
          pltpu.SemaphoreType.DMA,
      ),
  )
  def body(bag_hbm, off_hbm, v_hbm, uw_hbm, vw_hbm,
           psum_hbm, pcnt_hbm, embv_hbm,
           off_v, idx_v, seg_v, vidx_v, rows_v, vrows_v, ones_v,
           stage2d_v, stage1d_v, acc_s, cnt_s, sem):
    c = lax.axis_index("c")
    s = lax.axis_index("s")
    wid = s * NC + c

    zf = jnp.zeros((16,), jnp.float32)
    onef = jnp.full((16,), 1.0, jnp.float32)

    pltpu.sync_copy(off_hbm, off_v)
    pltpu.sync_copy(bag_hbm.at[pl.ds(wid * NCHUNK, NCHUNK)], idx_v)
    pltpu.sync_copy(v_hbm.at[pl.ds(wid * NVCHUNK, NVCHUNK)], vidx_v)

    for i in range(SEG_PER_TILE // 16):
      stage1d_v[pl.ds(i * 16, 16)] = zf

    def zrow(i, carry):
      stage2d_v[i, pl.ds(0, 16)] = zf
      stage2d_v[i, pl.ds(16, 16)] = zf
      return carry
    lax.fori_loop(0, 256, zrow, 0)

    for i in range(CHUNK // 16):
      ones_v[pl.ds(i * 16, 16)] = onef

    pltpu.sync_copy(stage1d_v, cnt_s.at[pl.ds(s * SEG_PER_TILE, SEG_PER_TILE)])
    for k in range(SEG_PER_TILE // 256):
      pltpu.sync_copy(stage2d_v,
                      acc_s.at[pl.ds(s * SEG_PER_TILE + k * 256, 256)])

    lane = lax.iota(jnp.int32, 16)

    def seg_chunk(j, carry):
      base = wid * POS_PER_W + j * CHUNK
      for k in range(CHUNK // 16):
        pos = base + k * 16 + lane
        lo = jnp.zeros((16,), jnp.int32)
        sz = B // 2
        while sz >= 1:
          cand = lo + sz
          oc = plsc.load_gather(off_v, [cand])
          lo = jnp.where(oc <= pos, cand, lo)
          sz //= 2
        seg_v[j, pl.ds(k * 16, 16)] = lo
      return carry
    lax.fori_loop(0, NCHUNK, seg_chunk, 0)

    plsc.subcore_barrier()

    def bag_chunk(j, carry):
      pltpu.async_copy(uw_hbm.at[idx_v.at[j]], rows_v, sem).wait()
      pltpu.sync_copy(rows_v, acc_s.at[seg_v.at[j]], add=True)
      pltpu.sync_copy(ones_v, cnt_s.at[seg_v.at[j]], add=True)
      return carry
    lax.fori_loop(0, NCHUNK, bag_chunk, 0)

    def v_chunk(j, carry):
      pltpu.async_copy(vw_hbm.at[vidx_v.at[j]], vrows_v, sem).wait()
      pltpu.sync_copy(vrows_v, embv_hbm.at[pl.ds(wid * V_PER_W + j * CHUNK,
                                                 CHUNK)])
      return carry
    lax.fori_loop(0, NVCHUNK, v_chunk, 0)

    plsc.subcore_barrier()

    pltpu.sync_copy(cnt_s.at[pl.ds(s * SEG_PER_TILE, SEG_PER_TILE)], stage1d_v)
    pltpu.sync_copy(stage1d_v, pcnt_hbm.at[c, pl.ds(s * SEG_PER_TILE,
                                                    SEG_PER_TILE)])
    for k in range(SEG_PER_TILE // 256):
      off0 = s * SEG_PER_TILE + k * 256
      pltpu.sync_copy(acc_s.at[pl.ds(off0, 256)], stage2d_v)
      pltpu.sync_copy(stage2d_v, psum_hbm.at[c, pl.ds(off0, 256)])

  return body(bag2d, offsets, v2d, u_weight, v_weight)


BLK = 1024


def _tc_body(ps_ref, pc_ref, ev_ref, out_ref):
  ps = ps_ref[...]
  pc = pc_ref[...]
  ev = ev_ref[...]
  sums = ps[0] + ps[1]
  cnt = pc[0] + pc[1]
  emb_u = sums / jnp.maximum(cnt, 1.0)[:, None]
  out_ref[...] = jnp.sum(emb_u[:, None, :] * ev, axis=-1)


def _tc_dot(psum, pcnt, embv3):
  nblk = B // BLK
  return pl.pallas_call(
      _tc_body,
      grid=(nblk,),
      in_specs=[
          pl.BlockSpec((NC, BLK, D), lambda i: (0, i, 0)),
          pl.BlockSpec((NC, BLK), lambda i: (0, i)),
          pl.BlockSpec((BLK, NSAMP, D), lambda i: (i, 0, 0)),
      ],
      out_specs=pl.BlockSpec((BLK, NSAMP), lambda i: (i, 0)),
      out_shape=jax.ShapeDtypeStruct((B, NSAMP), jnp.float32),
  )(psum, pcnt, embv3)


@jax.jit
def kernel(bag, offsets, v, u_weight, v_weight):
  bag2d = bag.astype(jnp.int32).reshape(NW * NCHUNK, CHUNK)
  v2d = v.astype(jnp.int32).reshape(NW * NVCHUNK, CHUNK)
  psum, pcnt, embv = _sc_kernel(bag2d, offsets.astype(jnp.int32), v2d,
                                u_weight, v_weight)
  return _tc_dot(psum, pcnt, embv.reshape(B, NSAMP, D))
